# Initial kernel scaffold; baseline (speedup 1.0000x reference)
import jax
import jax.numpy as jnp
from jax.experimental import pallas as pl


def kernel(x, edge_index, etypes, W_lin, b_lin, W_ih, W_hh, b_ih, b_hh, W_cls, b_cls):
    raise NotImplementedError("write your pallas kernel here")


# speedup vs baseline: 29.9500x; 29.9500x over previous
import functools

import jax
import jax.numpy as jnp
from jax import lax
from jax.experimental import pallas as pl
from jax.experimental.pallas import tpu as pltpu
from jax.experimental.pallas import tpu_sc as plsc

N_NODES = 10000
N_EDGES = 320000
D = 128
N_ETYPES = 4
N_STEPS = 8

NC = 2
NS = 16
NW = NC * NS
EPW = N_EDGES // NW
CH = 125
NCHUNK = EPW // CH
RPT = 624
RPT_LAST = N_NODES - (NS - 1) * RPT
ZCH = 104

BN = 1000
NB = N_NODES // BN


def _ht_body(h_ref, w_ref, b_ref, out_ref):
    h = h_ref[...]
    for t in range(N_ETYPES):
        out_ref[t] = lax.dot_general(
            h, w_ref[t], (((1,), (1,)), ((), ())),
            preferred_element_type=jnp.float32) + b_ref[t]


_ht_call = pl.pallas_call(
    _ht_body,
    grid=(NB,),
    in_specs=[
        pl.BlockSpec((BN, D), lambda n: (n, 0)),
        pl.BlockSpec((N_ETYPES, D, D), lambda n: (0, 0, 0)),
        pl.BlockSpec((N_ETYPES, D), lambda n: (0, 0)),
    ],
    out_specs=pl.BlockSpec((N_ETYPES, BN, D), lambda n: (0, n, 0)),
    out_shape=jax.ShapeDtypeStruct((N_ETYPES, N_NODES, D), jnp.float32),
)


def _sc_body(tbl, ci, dsti, out, idx_v, dst_v, rows_v, acc, sem):
    c = lax.axis_index("c")
    s = lax.axis_index("s")
    wid = s * NC + c

    zero = jnp.zeros((16,), jnp.float32)

    def _zr(i, carry):
        r = i // (D // 16)
        col = (i % (D // 16)) * 16
        rows_v[r, pl.ds(col, 16)] = zero
        return carry

    lax.fori_loop(0, ZCH * (D // 16), _zr, 0)
    base = s * RPT
    for k in range(RPT // ZCH):
        pltpu.sync_copy(rows_v.at[pl.ds(0, ZCH)],
                        acc.at[pl.ds(base + k * ZCH, ZCH)])

    @pl.when(s == NS - 1)
    def _zero_tail():
        pltpu.sync_copy(rows_v.at[pl.ds(0, RPT_LAST - RPT)],
                        acc.at[pl.ds(base + RPT, RPT_LAST - RPT)])

    plsc.subcore_barrier()

    pltpu.sync_copy(ci.at[wid], idx_v)
    pltpu.sync_copy(dsti.at[wid], dst_v)

    def _edge_chunk(j, carry):
        pltpu.async_copy(tbl.at[idx_v.at[j]], rows_v, sem).wait()
        pltpu.sync_copy(rows_v, acc.at[dst_v.at[j]], add=True)
        return carry

    lax.fori_loop(0, NCHUNK, _edge_chunk, 0)
    plsc.subcore_barrier()

    @pl.when(s < NS - 1)
    def _wb():
        pltpu.sync_copy(acc.at[pl.ds(s * RPT, RPT)],
                        out.at[c, pl.ds(s * RPT, RPT)])

    @pl.when(s == NS - 1)
    def _wb_last():
        pltpu.sync_copy(acc.at[pl.ds(s * RPT, RPT_LAST)],
                        out.at[c, pl.ds(s * RPT, RPT_LAST)])


_sc_segsum = functools.partial(
    pl.kernel,
    out_type=jax.ShapeDtypeStruct((NC, N_NODES, D), jnp.float32),
    mesh=plsc.VectorSubcoreMesh(core_axis_name="c", subcore_axis_name="s"),
    scratch_types=[
        pltpu.VMEM((NCHUNK, CH), jnp.int32),
        pltpu.VMEM((NCHUNK, CH), jnp.int32),
        pltpu.VMEM((CH, D), jnp.float32),
        pltpu.VMEM_SHARED((N_NODES, D), jnp.float32),
        pltpu.SemaphoreType.DMA,
    ],
)(_sc_body)


def _gru_body(ap_ref, h_ref, wih_ref, whh_ref, bih_ref, bhh_ref, out_ref):
    a = ap_ref[0] + ap_ref[1]
    h = h_ref[...]
    gi = jnp.dot(a, wih_ref[...], preferred_element_type=jnp.float32) \
        + bih_ref[...]
    gh = jnp.dot(h, whh_ref[...], preferred_element_type=jnp.float32) \
        + bhh_ref[...]
    r = jax.nn.sigmoid(gi[:, :D] + gh[:, :D])
    z = jax.nn.sigmoid(gi[:, D:2 * D] + gh[:, D:2 * D])
    n = jnp.tanh(gi[:, 2 * D:] + r * gh[:, 2 * D:])
    out_ref[...] = (1.0 - z) * n + z * h


_gru_call = pl.pallas_call(
    _gru_body,
    grid=(NB,),
    in_specs=[
        pl.BlockSpec((NC, BN, D), lambda n: (0, n, 0)),
        pl.BlockSpec((BN, D), lambda n: (n, 0)),
        pl.BlockSpec((D, 3 * D), lambda n: (0, 0)),
        pl.BlockSpec((D, 3 * D), lambda n: (0, 0)),
        pl.BlockSpec((1, 3 * D), lambda n: (0, 0)),
        pl.BlockSpec((1, 3 * D), lambda n: (0, 0)),
    ],
    out_specs=pl.BlockSpec((BN, D), lambda n: (n, 0)),
    out_shape=jax.ShapeDtypeStruct((N_NODES, D), jnp.float32),
)


def _cls_body(h_ref, w_ref, b_ref, out_ref):
    i = pl.program_id(0)
    part = jnp.sum(jnp.sum(h_ref[...], axis=0) * w_ref[0])

    @pl.when(i == 0)
    def _init():
        out_ref[0, 0] = part

    @pl.when(i > 0)
    def _acc():
        out_ref[0, 0] = out_ref[0, 0] + part

    @pl.when(i == pl.num_programs(0) - 1)
    def _fin():
        out_ref[0, 0] = jax.nn.sigmoid(out_ref[0, 0] + b_ref[0, 0])


_cls_call = pl.pallas_call(
    _cls_body,
    grid=(NB,),
    in_specs=[
        pl.BlockSpec((BN, D), lambda n: (n, 0)),
        pl.BlockSpec((1, D), lambda n: (0, 0)),
        pl.BlockSpec((1, 1), lambda n: (0, 0)),
    ],
    out_specs=pl.BlockSpec((1, 1), lambda n: (0, 0),
                           memory_space=pltpu.SMEM),
    out_shape=jax.ShapeDtypeStruct((1, 1), jnp.float32),
)


def kernel(x, edge_index, etypes, W_lin, b_lin, W_ih, W_hh, b_ih, b_hh,
           W_cls, b_cls):
    src = edge_index[0]
    dst = edge_index[1]
    ci = (etypes * N_NODES + src).reshape(NW, NCHUNK, CH)
    dsti = dst.reshape(NW, NCHUNK, CH)

    wih_t = W_ih.T
    whh_t = W_hh.T
    bih2 = b_ih.reshape(1, 3 * D)
    bhh2 = b_hh.reshape(1, 3 * D)

    h = x
    for _ in range(N_STEPS):
        ht = _ht_call(h, W_lin, b_lin).reshape(N_ETYPES * N_NODES, D)
        aparts = _sc_segsum(ht, ci, dsti)
        h = _gru_call(aparts, h, wih_t, whh_t, bih2, bhh2)

    out = _cls_call(h, W_cls, b_cls.reshape(1, 1))
    return out.reshape(())


# speedup vs baseline: 33.5205x; 1.1192x over previous
import functools

import jax
import jax.numpy as jnp
from jax import lax
from jax.experimental import pallas as pl
from jax.experimental.pallas import tpu as pltpu
from jax.experimental.pallas import tpu_sc as plsc

N_NODES = 10000
N_EDGES = 320000
D = 128
N_ETYPES = 4
N_STEPS = 8

NC = 2
NS = 16
NW = NC * NS
EPW = N_EDGES // NW
CH = 125
NCHUNK = EPW // CH
GCH = 16
NGRP = NCHUNK // GCH
RPT = 624
RPT_LAST = N_NODES - (NS - 1) * RPT
ZCH = 48

BN = 1000
NB = N_NODES // BN


def _ht_body(h_ref, w_ref, b_ref, out_ref):
    h = h_ref[...]
    for t in range(N_ETYPES):
        out_ref[t] = lax.dot_general(
            h, w_ref[t], (((1,), (1,)), ((), ())),
            preferred_element_type=jnp.float32) + b_ref[t]


_ht_call = pl.pallas_call(
    _ht_body,
    grid=(NB,),
    in_specs=[
        pl.BlockSpec((BN, D), lambda n: (n, 0)),
        pl.BlockSpec((N_ETYPES, D, D), lambda n: (0, 0, 0)),
        pl.BlockSpec((N_ETYPES, D), lambda n: (0, 0)),
    ],
    out_specs=pl.BlockSpec((N_ETYPES, BN, D), lambda n: (0, n, 0)),
    out_shape=jax.ShapeDtypeStruct((N_ETYPES, N_NODES, D), jnp.float32),
)


def _sc_body(tbl, ci, dsti, out, idx_v, dst_v, rows_v, rows_w, acc, sem,
             sem_b):
    c = lax.axis_index("c")
    s = lax.axis_index("s")
    wid = s * NC + c

    zero = jnp.zeros((16,), jnp.float32)

    def _zr(i, carry):
        r = i // (D // 16)
        col = (i % (D // 16)) * 16
        rows_v[r, pl.ds(col, 16)] = zero
        return carry

    lax.fori_loop(0, ZCH * (D // 16), _zr, 0)
    base = s * RPT
    for k in range(RPT // ZCH):
        pltpu.sync_copy(rows_v.at[pl.ds(0, ZCH)],
                        acc.at[pl.ds(base + k * ZCH, ZCH)])

    @pl.when(s == NS - 1)
    def _zero_tail():
        pltpu.sync_copy(rows_v.at[pl.ds(0, RPT_LAST - RPT)],
                        acc.at[pl.ds(base + RPT, RPT_LAST - RPT)])

    plsc.subcore_barrier()

    def _group(g, carry):
        pltpu.sync_copy(ci.at[wid, pl.ds(g * GCH, GCH)], idx_v)
        pltpu.sync_copy(dsti.at[wid, pl.ds(g * GCH, GCH)], dst_v)

        def _edge_chunk(k, carry2):
            j = 2 * k
            ca = pltpu.async_copy(tbl.at[idx_v.at[j]], rows_v, sem)
            cb = pltpu.async_copy(tbl.at[idx_v.at[j + 1]], rows_w, sem_b)
            ca.wait()
            pltpu.sync_copy(rows_v, acc.at[dst_v.at[j]], add=True)
            cb.wait()
            pltpu.sync_copy(rows_w, acc.at[dst_v.at[j + 1]], add=True)
            return carry2

        lax.fori_loop(0, GCH // 2, _edge_chunk, 0)
        return carry

    lax.fori_loop(0, NGRP, _group, 0)
    plsc.subcore_barrier()

    @pl.when(s < NS - 1)
    def _wb():
        pltpu.sync_copy(acc.at[pl.ds(s * RPT, RPT)],
                        out.at[c, pl.ds(s * RPT, RPT)])

    @pl.when(s == NS - 1)
    def _wb_last():
        pltpu.sync_copy(acc.at[pl.ds(s * RPT, RPT_LAST)],
                        out.at[c, pl.ds(s * RPT, RPT_LAST)])


_sc_segsum = functools.partial(
    pl.kernel,
    out_type=jax.ShapeDtypeStruct((NC, N_NODES, D), jnp.float32),
    mesh=plsc.VectorSubcoreMesh(core_axis_name="c", subcore_axis_name="s"),
    scratch_types=[
        pltpu.VMEM((GCH, CH), jnp.int32),
        pltpu.VMEM((GCH, CH), jnp.int32),
        pltpu.VMEM((CH, D), jnp.float32),
        pltpu.VMEM((CH, D), jnp.float32),
        pltpu.VMEM_SHARED((N_NODES, D), jnp.float32),
        pltpu.SemaphoreType.DMA,
        pltpu.SemaphoreType.DMA,
    ],
)(_sc_body)


def _gru_body(ap_ref, h_ref, wih_ref, whh_ref, bih_ref, bhh_ref, out_ref):
    a = ap_ref[0] + ap_ref[1]
    h = h_ref[...]
    gi = jnp.dot(a, wih_ref[...], preferred_element_type=jnp.float32) \
        + bih_ref[...]
    gh = jnp.dot(h, whh_ref[...], preferred_element_type=jnp.float32) \
        + bhh_ref[...]
    r = jax.nn.sigmoid(gi[:, :D] + gh[:, :D])
    z = jax.nn.sigmoid(gi[:, D:2 * D] + gh[:, D:2 * D])
    n = jnp.tanh(gi[:, 2 * D:] + r * gh[:, 2 * D:])
    out_ref[...] = (1.0 - z) * n + z * h


_gru_call = pl.pallas_call(
    _gru_body,
    grid=(NB,),
    in_specs=[
        pl.BlockSpec((NC, BN, D), lambda n: (0, n, 0)),
        pl.BlockSpec((BN, D), lambda n: (n, 0)),
        pl.BlockSpec((D, 3 * D), lambda n: (0, 0)),
        pl.BlockSpec((D, 3 * D), lambda n: (0, 0)),
        pl.BlockSpec((1, 3 * D), lambda n: (0, 0)),
        pl.BlockSpec((1, 3 * D), lambda n: (0, 0)),
    ],
    out_specs=pl.BlockSpec((BN, D), lambda n: (n, 0)),
    out_shape=jax.ShapeDtypeStruct((N_NODES, D), jnp.float32),
)


def _cls_body(h_ref, w_ref, b_ref, out_ref):
    i = pl.program_id(0)
    part = jnp.sum(jnp.sum(h_ref[...], axis=0) * w_ref[0])

    @pl.when(i == 0)
    def _init():
        out_ref[0, 0] = part

    @pl.when(i > 0)
    def _acc():
        out_ref[0, 0] = out_ref[0, 0] + part

    @pl.when(i == pl.num_programs(0) - 1)
    def _fin():
        out_ref[0, 0] = jax.nn.sigmoid(out_ref[0, 0] + b_ref[0, 0])


_cls_call = pl.pallas_call(
    _cls_body,
    grid=(NB,),
    in_specs=[
        pl.BlockSpec((BN, D), lambda n: (n, 0)),
        pl.BlockSpec((1, D), lambda n: (0, 0)),
        pl.BlockSpec((1, 1), lambda n: (0, 0)),
    ],
    out_specs=pl.BlockSpec((1, 1), lambda n: (0, 0),
                           memory_space=pltpu.SMEM),
    out_shape=jax.ShapeDtypeStruct((1, 1), jnp.float32),
)


def kernel(x, edge_index, etypes, W_lin, b_lin, W_ih, W_hh, b_ih, b_hh,
           W_cls, b_cls):
    src = edge_index[0]
    dst = edge_index[1]
    ci = (etypes * N_NODES + src).reshape(NW, NCHUNK, CH)
    dsti = dst.reshape(NW, NCHUNK, CH)

    wih_t = W_ih.T
    whh_t = W_hh.T
    bih2 = b_ih.reshape(1, 3 * D)
    bhh2 = b_hh.reshape(1, 3 * D)

    h = x
    for _ in range(N_STEPS):
        ht = _ht_call(h, W_lin, b_lin).reshape(N_ETYPES * N_NODES, D)
        aparts = _sc_segsum(ht, ci, dsti)
        h = _gru_call(aparts, h, wih_t, whh_t, bih2, bhh2)

    out = _cls_call(h, W_cls, b_cls.reshape(1, 1))
    return out.reshape(())
